# baseline (speedup 1.0000x reference)
import functools

import jax
import jax.numpy as jnp
from jax import lax
from jax.experimental import pallas as pl
from jax.experimental.pallas import tpu as pltpu
from jax.experimental.pallas import tpu_sc as plsc

N_PAIRS = 320000
N_ATOMS = 10000
N_MOL = 512
D = 128
NC = 2
NS = 16
NW = NC * NS
L = 16

SC_PAIRS = 215040
PT = SC_PAIRS // NW
C = 80
NCHUNK = PT // C
NB = 6

BP = 2560
TC_BLK0 = SC_PAIRS // BP
TC_NBLK = (N_PAIRS - SC_PAIRS) // BP
MI_ROWS = 80


def _sc_segment_sum(pairfeatures, mol_index, pair_first, zeros):
    mesh = plsc.VectorSubcoreMesh(core_axis_name="c", subcore_axis_name="s")

    @functools.partial(
        pl.kernel,
        mesh=mesh,
        out_type=jax.ShapeDtypeStruct((NC, N_MOL, D), jnp.float32),
        compiler_params=pltpu.CompilerParams(needs_layout_passes=False),
        scratch_types=[
            pltpu.VMEM((PT,), jnp.int32),
            pltpu.VMEM((N_ATOMS,), jnp.int32),
            pltpu.VMEM((NCHUNK, C), jnp.int32),
            *[pltpu.VMEM((C, D), jnp.float32) for _ in range(NB)],
            pltpu.VMEM_SHARED((N_MOL, D), jnp.float32),
            *[pltpu.SemaphoreType.DMA for _ in range(2 * NB)],
        ],
    )
    def seg_sum(feat_hbm, mi_hbm, pf_hbm, z_hbm, out_hbm,
                pf_v, mi_v, pm_v, fv0, fv1, fv2, fv3, fv4, fv5, acc_sh,
                li0, li1, li2, li3, li4, li5, ai0, ai1, ai2, ai3, ai4, ai5):
        core = lax.axis_index("c")
        sub = lax.axis_index("s")
        wid = core * NS + sub
        base = wid * PT
        bufs = (fv0, fv1, fv2, fv3, fv4, fv5)
        lsems = (li0, li1, li2, li3, li4, li5)
        asems = (ai0, ai1, ai2, ai3, ai4, ai5)

        def start_load(j, b):
            pltpu.async_copy(feat_hbm.at[pl.ds(base + j * C, C)], bufs[b], lsems[b])

        def wait_load(b):
            pltpu.make_async_copy(feat_hbm.at[pl.ds(0, C)], bufs[b], lsems[b]).wait()

        def start_add(j, b):
            pltpu.async_copy(bufs[b], acc_sh.at[pm_v.at[j]], asems[b], add=True)

        def wait_add(j, b):
            pltpu.make_async_copy(bufs[b], acc_sh.at[pm_v.at[j]], asems[b]).wait()

        for _j in range(NB - 1):
            start_load(_j, _j)

        pltpu.sync_copy(pf_hbm.at[pl.ds(base, PT)], pf_v)
        pltpu.sync_copy(mi_hbm, mi_v)

        @pl.when(sub == 0)
        def _():
            pltpu.sync_copy(z_hbm, acc_sh)

        def gather_body(j, carry):
            r0 = j * C
            for k in range(C // L):
                idx = pf_v[pl.ds(r0 + k * L, L)]
                pm_v[j, pl.ds(k * L, L)] = plsc.load_gather(mi_v, [idx])
            return carry

        lax.fori_loop(0, NCHUNK, gather_body, 0)

        plsc.subcore_barrier()

        def add_body(jj, carry):
            for b in range(NB):
                j = NB * jj + b
                wait_load(b)
                start_add(j, b)

                @pl.when(j >= 1)
                def _():
                    wait_add(j - 1, (b - 1) % NB)

                @pl.when(j + (NB - 1) < NCHUNK)
                def _():
                    start_load(j + (NB - 1), (b + NB - 1) % NB)

            return carry

        NFULL = NCHUNK // NB
        lax.fori_loop(0, NFULL, add_body, 0)

        for j in range(NFULL * NB, NCHUNK):
            b = j % NB
            wait_load(b)
            start_add(j, b)
            wait_add(j - 1, (b - 1) % NB)
        wait_add(NCHUNK - 1, (NCHUNK - 1) % NB)

        plsc.subcore_barrier()

        rows = N_MOL // NS
        pltpu.sync_copy(acc_sh.at[pl.ds(sub * rows, rows)],
                        out_hbm.at[core, pl.ds(sub * rows, rows)])

    return seg_sum(pairfeatures, mol_index, pair_first, zeros)


def _tc_partial(mi_pad, pf_blocks, pairfeatures):
    def body(mi_ref, pf_ref, feat_ref, out_ref, lo_ref, hi_ref):
        i = pl.program_id(0)

        @pl.when(i == 0)
        def _():
            m = lax.broadcasted_iota(jnp.int32, (N_MOL, 128), 0)

            def row(r, carry):
                lo, hi = carry
                a = mi_ref[pl.ds(r, 1), :]
                lo = lo + jnp.sum((a < m).astype(jnp.int32), axis=1,
                                  keepdims=True)
                hi = hi + jnp.sum((a <= m).astype(jnp.int32), axis=1,
                                  keepdims=True)
                return lo, hi

            lo0 = jnp.zeros((N_MOL, 1), jnp.int32)
            hi0 = jnp.zeros((N_MOL, 1), jnp.int32)
            lo, hi = lax.fori_loop(0, MI_ROWS, row, (lo0, hi0))
            lo_ref[...] = lo
            hi_ref[...] = hi
            out_ref[...] = jnp.zeros_like(out_ref)

        pfb = pf_ref[0].astype(jnp.int16)
        lo = lo_ref[...].astype(jnp.int16)
        hi = hi_ref[...].astype(jnp.int16)
        oh = ((pfb >= lo) & (pfb < hi)).astype(jnp.bfloat16)
        fb = feat_ref[...].astype(jnp.bfloat16)
        out_ref[...] += lax.dot_general(
            oh, fb, (((1,), (0,)), ((), ())),
            preferred_element_type=jnp.float32)

    return pl.pallas_call(
        body,
        grid=(TC_NBLK,),
        in_specs=[
            pl.BlockSpec((MI_ROWS, 128), lambda i: (0, 0)),
            pl.BlockSpec((1, 1, BP), lambda i: (TC_BLK0 + i, 0, 0)),
            pl.BlockSpec((BP, D), lambda i: (TC_BLK0 + i, 0)),
        ],
        out_specs=pl.BlockSpec((N_MOL, D), lambda i: (0, 0)),
        out_shape=jax.ShapeDtypeStruct((N_MOL, D), jnp.float32),
        scratch_shapes=[
            pltpu.VMEM((N_MOL, 1), jnp.int32),
            pltpu.VMEM((N_MOL, 1), jnp.int32),
        ],
    )(mi_pad, pf_blocks, pairfeatures)


def _combine(sc_partials, tc_partial):
    def body(p_ref, t_ref, o_ref):
        o_ref[...] = p_ref[0] + p_ref[1] + t_ref[...]

    return pl.pallas_call(
        body,
        out_shape=jax.ShapeDtypeStruct((N_MOL, D), jnp.float32),
    )(sc_partials, tc_partial)


def kernel(pairfeatures, mol_index, n_molecules, pair_first):
    mi32 = mol_index.astype(jnp.int32)
    pf32 = pair_first.astype(jnp.int32)
    zeros = jnp.zeros((N_MOL, D), dtype=jnp.float32)
    mi_pad = jnp.concatenate(
        [mi32, jnp.full((MI_ROWS * 128 - N_ATOMS,), jnp.int32(1 << 30))]
    ).reshape(MI_ROWS, 128)

    pf_blocks = pf32.reshape(N_PAIRS // BP, 1, BP)
    sc_partials = _sc_segment_sum(pairfeatures, mi32, pf32, zeros)
    tc_partial = _tc_partial(mi_pad, pf_blocks, pairfeatures)
    return _combine(sc_partials, tc_partial)


# speedup vs baseline: 1.1015x; 1.1015x over previous
import functools

import jax
import jax.numpy as jnp
from jax import lax
from jax.experimental import pallas as pl
from jax.experimental.pallas import tpu as pltpu
from jax.experimental.pallas import tpu_sc as plsc

N_PAIRS = 320000
N_ATOMS = 10000
N_MOL = 512
D = 128
NC = 2
NS = 16
NW = NC * NS
L = 16

SC_PAIRS = 238080
PT = SC_PAIRS // NW
C = 80
NCHUNK = PT // C
NB = 6

BP = 2560
TC_BLK0 = SC_PAIRS // BP
TC_NBLK = (N_PAIRS - SC_PAIRS) // BP
MI_ROWS = 80


def _sc_segment_sum(pairfeatures, mol_index, pair_first, zeros):
    mesh = plsc.VectorSubcoreMesh(core_axis_name="c", subcore_axis_name="s")

    @functools.partial(
        pl.kernel,
        mesh=mesh,
        out_type=jax.ShapeDtypeStruct((NC, N_MOL, D), jnp.float32),
        compiler_params=pltpu.CompilerParams(needs_layout_passes=False),
        scratch_types=[
            pltpu.VMEM((PT,), jnp.int32),
            pltpu.VMEM((N_ATOMS,), jnp.int32),
            pltpu.VMEM((NCHUNK, C), jnp.int32),
            *[pltpu.VMEM((C, D), jnp.float32) for _ in range(NB)],
            pltpu.VMEM_SHARED((N_MOL, D), jnp.float32),
            *[pltpu.SemaphoreType.DMA for _ in range(2 * NB)],
        ],
    )
    def seg_sum(feat_hbm, mi_hbm, pf_hbm, z_hbm, out_hbm,
                pf_v, mi_v, pm_v, fv0, fv1, fv2, fv3, fv4, fv5, acc_sh,
                li0, li1, li2, li3, li4, li5, ai0, ai1, ai2, ai3, ai4, ai5):
        core = lax.axis_index("c")
        sub = lax.axis_index("s")
        wid = core * NS + sub
        base = wid * PT
        bufs = (fv0, fv1, fv2, fv3, fv4, fv5)
        lsems = (li0, li1, li2, li3, li4, li5)
        asems = (ai0, ai1, ai2, ai3, ai4, ai5)

        def start_load(j, b):
            pltpu.async_copy(feat_hbm.at[pl.ds(base + j * C, C)], bufs[b], lsems[b])

        def wait_load(b):
            pltpu.make_async_copy(feat_hbm.at[pl.ds(0, C)], bufs[b], lsems[b]).wait()

        def start_add(j, b):
            pltpu.async_copy(bufs[b], acc_sh.at[pm_v.at[j]], asems[b], add=True)

        def wait_add(j, b):
            pltpu.make_async_copy(bufs[b], acc_sh.at[pm_v.at[j]], asems[b]).wait()

        for _j in range(NB - 1):
            start_load(_j, _j)

        pltpu.sync_copy(pf_hbm.at[pl.ds(base, PT)], pf_v)
        pltpu.sync_copy(mi_hbm, mi_v)

        @pl.when(sub == 0)
        def _():
            pltpu.sync_copy(z_hbm, acc_sh)

        def gather_body(j, carry):
            r0 = j * C
            for k in range(C // L):
                idx = pf_v[pl.ds(r0 + k * L, L)]
                pm_v[j, pl.ds(k * L, L)] = plsc.load_gather(mi_v, [idx])
            return carry

        lax.fori_loop(0, NCHUNK, gather_body, 0)

        plsc.subcore_barrier()

        def add_body(jj, carry):
            for b in range(NB):
                j = NB * jj + b
                wait_load(b)
                start_add(j, b)

                @pl.when(j >= 1)
                def _():
                    wait_add(j - 1, (b - 1) % NB)

                @pl.when(j + (NB - 1) < NCHUNK)
                def _():
                    start_load(j + (NB - 1), (b + NB - 1) % NB)

            return carry

        NFULL = NCHUNK // NB
        lax.fori_loop(0, NFULL, add_body, 0)

        for j in range(NFULL * NB, NCHUNK):
            b = j % NB
            wait_load(b)
            start_add(j, b)
            wait_add(j - 1, (b - 1) % NB)
        wait_add(NCHUNK - 1, (NCHUNK - 1) % NB)

        plsc.subcore_barrier()

        rows = N_MOL // NS
        pltpu.sync_copy(acc_sh.at[pl.ds(sub * rows, rows)],
                        out_hbm.at[core, pl.ds(sub * rows, rows)])

    return seg_sum(pairfeatures, mol_index, pair_first, zeros)


def _tc_partial(mi_pad, pf_blocks, pairfeatures):
    def body(mi_ref, pf_ref, feat_ref, out_ref, lo_ref, hi_ref):
        i = pl.program_id(0)

        @pl.when(i == 0)
        def _():
            m = lax.broadcasted_iota(jnp.int32, (N_MOL, 128), 0)

            def row(r, carry):
                lo, hi = carry
                a = mi_ref[pl.ds(r, 1), :]
                lo = lo + jnp.sum((a < m).astype(jnp.int32), axis=1,
                                  keepdims=True)
                hi = hi + jnp.sum((a <= m).astype(jnp.int32), axis=1,
                                  keepdims=True)
                return lo, hi

            lo0 = jnp.zeros((N_MOL, 1), jnp.int32)
            hi0 = jnp.zeros((N_MOL, 1), jnp.int32)
            lo, hi = lax.fori_loop(0, MI_ROWS, row, (lo0, hi0))
            lo_ref[...] = lo
            hi_ref[...] = hi
            out_ref[...] = jnp.zeros_like(out_ref)

        pfb = pf_ref[0].astype(jnp.int16)
        lo = lo_ref[...].astype(jnp.int16)
        hi = hi_ref[...].astype(jnp.int16)
        oh = ((pfb >= lo) & (pfb < hi)).astype(jnp.bfloat16)
        fb = feat_ref[...].astype(jnp.bfloat16)
        out_ref[...] += lax.dot_general(
            oh, fb, (((1,), (0,)), ((), ())),
            preferred_element_type=jnp.float32)

    return pl.pallas_call(
        body,
        grid=(TC_NBLK,),
        in_specs=[
            pl.BlockSpec((MI_ROWS, 128), lambda i: (0, 0)),
            pl.BlockSpec((1, 1, BP), lambda i: (TC_BLK0 + i, 0, 0)),
            pl.BlockSpec((BP, D), lambda i: (TC_BLK0 + i, 0)),
        ],
        out_specs=pl.BlockSpec((N_MOL, D), lambda i: (0, 0)),
        out_shape=jax.ShapeDtypeStruct((N_MOL, D), jnp.float32),
        scratch_shapes=[
            pltpu.VMEM((N_MOL, 1), jnp.int32),
            pltpu.VMEM((N_MOL, 1), jnp.int32),
        ],
    )(mi_pad, pf_blocks, pairfeatures)


def _combine(sc_partials, tc_partial):
    def body(p_ref, t_ref, o_ref):
        o_ref[...] = p_ref[0] + p_ref[1] + t_ref[...]

    return pl.pallas_call(
        body,
        out_shape=jax.ShapeDtypeStruct((N_MOL, D), jnp.float32),
    )(sc_partials, tc_partial)


def kernel(pairfeatures, mol_index, n_molecules, pair_first):
    mi32 = mol_index.astype(jnp.int32)
    pf32 = pair_first.astype(jnp.int32)
    zeros = jnp.zeros((N_MOL, D), dtype=jnp.float32)
    mi_pad = jnp.concatenate(
        [mi32, jnp.full((MI_ROWS * 128 - N_ATOMS,), jnp.int32(1 << 30))]
    ).reshape(MI_ROWS, 128)

    pf_blocks = pf32.reshape(N_PAIRS // BP, 1, BP)
    sc_partials = _sc_segment_sum(pairfeatures, mi32, pf32, zeros)
    tc_partial = _tc_partial(mi_pad, pf_blocks, pairfeatures)
    return _combine(sc_partials, tc_partial)


# speedup vs baseline: 1.1568x; 1.0502x over previous
import functools

import jax
import jax.numpy as jnp
from jax import lax
from jax.experimental import pallas as pl
from jax.experimental.pallas import tpu as pltpu
from jax.experimental.pallas import tpu_sc as plsc

N_PAIRS = 320000
N_ATOMS = 10000
N_MOL = 512
D = 128
NC = 2
NS = 16
NW = NC * NS
L = 16

SC_PAIRS = 238080
PT = SC_PAIRS // NW
C = 80
NCHUNK = PT // C
NB = 6

BP = 2560
TC_BLK0 = SC_PAIRS // BP
TC_NBLK = (N_PAIRS - SC_PAIRS) // BP


def _sc_segment_sum(pairfeatures, mol_index, pair_first, zeros):
    mesh = plsc.VectorSubcoreMesh(core_axis_name="c", subcore_axis_name="s")

    @functools.partial(
        pl.kernel,
        mesh=mesh,
        out_type=jax.ShapeDtypeStruct((NC, N_MOL, D), jnp.float32),
        compiler_params=pltpu.CompilerParams(needs_layout_passes=False),
        scratch_types=[
            pltpu.VMEM((PT,), jnp.int32),
            pltpu.VMEM((N_ATOMS,), jnp.int32),
            pltpu.VMEM((NCHUNK, C), jnp.int32),
            *[pltpu.VMEM((C, D), jnp.float32) for _ in range(NB)],
            pltpu.VMEM_SHARED((N_MOL, D), jnp.float32),
            *[pltpu.SemaphoreType.DMA for _ in range(2 * NB)],
        ],
    )
    def seg_sum(feat_hbm, mi_hbm, pf_hbm, z_hbm, out_hbm,
                pf_v, mi_v, pm_v, fv0, fv1, fv2, fv3, fv4, fv5, acc_sh,
                li0, li1, li2, li3, li4, li5, ai0, ai1, ai2, ai3, ai4, ai5):
        core = lax.axis_index("c")
        sub = lax.axis_index("s")
        wid = core * NS + sub
        base = wid * PT
        bufs = (fv0, fv1, fv2, fv3, fv4, fv5)
        lsems = (li0, li1, li2, li3, li4, li5)
        asems = (ai0, ai1, ai2, ai3, ai4, ai5)

        def start_load(j, b):
            pltpu.async_copy(feat_hbm.at[pl.ds(base + j * C, C)], bufs[b], lsems[b])

        def wait_load(b):
            pltpu.make_async_copy(feat_hbm.at[pl.ds(0, C)], bufs[b], lsems[b]).wait()

        def start_add(j, b):
            pltpu.async_copy(bufs[b], acc_sh.at[pm_v.at[j]], asems[b], add=True)

        def wait_add(j, b):
            pltpu.make_async_copy(bufs[b], acc_sh.at[pm_v.at[j]], asems[b]).wait()

        for _j in range(NB - 1):
            start_load(_j, _j)

        pltpu.sync_copy(pf_hbm.at[pl.ds(base, PT)], pf_v)
        pltpu.sync_copy(mi_hbm, mi_v)

        @pl.when(sub == 0)
        def _():
            pltpu.sync_copy(z_hbm, acc_sh)

        def gather_body(j, carry):
            r0 = j * C
            for k in range(C // L):
                idx = pf_v[pl.ds(r0 + k * L, L)]
                pm_v[j, pl.ds(k * L, L)] = plsc.load_gather(mi_v, [idx])
            return carry

        lax.fori_loop(0, NCHUNK, gather_body, 0)

        plsc.subcore_barrier()

        def add_body(jj, carry):
            for b in range(NB):
                j = NB * jj + b
                wait_load(b)
                start_add(j, b)

                @pl.when(j >= 1)
                def _():
                    wait_add(j - 1, (b - 1) % NB)

                @pl.when(j + (NB - 1) < NCHUNK)
                def _():
                    start_load(j + (NB - 1), (b + NB - 1) % NB)

            return carry

        NFULL = NCHUNK // NB
        lax.fori_loop(0, NFULL, add_body, 0)

        for j in range(NFULL * NB, NCHUNK):
            b = j % NB
            wait_load(b)
            start_add(j, b)
            wait_add(j - 1, (b - 1) % NB)
        wait_add(NCHUNK - 1, (NCHUNK - 1) % NB)

        plsc.subcore_barrier()

        rows = N_MOL // NS
        pltpu.sync_copy(acc_sh.at[pl.ds(sub * rows, rows)],
                        out_hbm.at[core, pl.ds(sub * rows, rows)])

    return seg_sum(pairfeatures, mol_index, pair_first, zeros)


def _tc_partial(mol_index, pair_first, pairfeatures):
    def body(mi_ref, pf_ref, feat_ref, out_ref, lo_ref, hi_ref):
        i = pl.program_id(0)

        @pl.when(i == 0)
        def _():
            m = lax.broadcasted_iota(jnp.int32, (N_MOL, 128), 0)

            def row(r, carry):
                lo, hi = carry
                a = mi_ref[pl.ds(pl.multiple_of(r * 128, 128), 128)]
                a = a.reshape(1, 128)
                lo = lo + jnp.sum((a < m).astype(jnp.int32), axis=1,
                                  keepdims=True)
                hi = hi + jnp.sum((a <= m).astype(jnp.int32), axis=1,
                                  keepdims=True)
                return lo, hi

            lo0 = jnp.zeros((N_MOL, 1), jnp.int32)
            hi0 = jnp.zeros((N_MOL, 1), jnp.int32)
            lo, hi = lax.fori_loop(0, N_ATOMS // 128, row, (lo0, hi0))
            mt = lax.broadcasted_iota(jnp.int32, (N_MOL, 16), 0)
            at = mi_ref[pl.ds(N_ATOMS - 16, 16)].reshape(1, 16)
            lo = lo + jnp.sum((at < mt).astype(jnp.int32), axis=1,
                              keepdims=True)
            hi = hi + jnp.sum((at <= mt).astype(jnp.int32), axis=1,
                              keepdims=True)
            lo_ref[...] = lo
            hi_ref[...] = hi
            out_ref[...] = jnp.zeros_like(out_ref)

        off = pl.multiple_of((TC_BLK0 + i) * BP, BP)
        pfb = pf_ref[pl.ds(off, BP)].reshape(1, BP).astype(jnp.int16)
        lo = lo_ref[...].astype(jnp.int16)
        hi = hi_ref[...].astype(jnp.int16)
        oh = ((pfb >= lo) & (pfb < hi)).astype(jnp.bfloat16)
        fb = feat_ref[...].astype(jnp.bfloat16)
        out_ref[...] += lax.dot_general(
            oh, fb, (((1,), (0,)), ((), ())),
            preferred_element_type=jnp.float32)

    return pl.pallas_call(
        body,
        grid=(TC_NBLK,),
        in_specs=[
            pl.BlockSpec((N_ATOMS,), lambda i: (0,)),
            pl.BlockSpec((N_PAIRS,), lambda i: (0,)),
            pl.BlockSpec((BP, D), lambda i: (TC_BLK0 + i, 0)),
        ],
        out_specs=pl.BlockSpec((N_MOL, D), lambda i: (0, 0)),
        out_shape=jax.ShapeDtypeStruct((N_MOL, D), jnp.float32),
        scratch_shapes=[
            pltpu.VMEM((N_MOL, 1), jnp.int32),
            pltpu.VMEM((N_MOL, 1), jnp.int32),
        ],
    )(mol_index, pair_first, pairfeatures)


def _combine(sc_partials, tc_partial):
    def body(p_ref, t_ref, o_ref):
        o_ref[...] = p_ref[0] + p_ref[1] + t_ref[...]

    return pl.pallas_call(
        body,
        out_shape=jax.ShapeDtypeStruct((N_MOL, D), jnp.float32),
    )(sc_partials, tc_partial)


def kernel(pairfeatures, mol_index, n_molecules, pair_first):
    mi32 = mol_index.astype(jnp.int32)
    pf32 = pair_first.astype(jnp.int32)
    zeros = jnp.zeros((N_MOL, D), dtype=jnp.float32)
    sc_partials = _sc_segment_sum(pairfeatures, mi32, pf32, zeros)
    tc_partial = _tc_partial(mi32, pf32, pairfeatures)
    return _combine(sc_partials, tc_partial)


# speedup vs baseline: 1.1846x; 1.0240x over previous
import functools

import jax
import jax.numpy as jnp
from jax import lax
from jax.experimental import pallas as pl
from jax.experimental.pallas import tpu as pltpu
from jax.experimental.pallas import tpu_sc as plsc

N_PAIRS = 320000
N_ATOMS = 10000
N_MOL = 512
D = 128
NC = 2
NS = 16
NW = NC * NS
L = 16

SC_PAIRS = 243200
PT = SC_PAIRS // NW
C = 80
NCHUNK = PT // C
NB = 6

BP = 2560
TC_BLK0 = SC_PAIRS // BP
TC_NBLK = (N_PAIRS - SC_PAIRS) // BP


def _sc_segment_sum(pairfeatures, mol_index, pair_first, zeros):
    mesh = plsc.VectorSubcoreMesh(core_axis_name="c", subcore_axis_name="s")

    @functools.partial(
        pl.kernel,
        mesh=mesh,
        out_type=jax.ShapeDtypeStruct((NC, N_MOL, D), jnp.float32),
        compiler_params=pltpu.CompilerParams(needs_layout_passes=False),
        scratch_types=[
            pltpu.VMEM((PT,), jnp.int32),
            pltpu.VMEM((N_ATOMS,), jnp.int32),
            pltpu.VMEM((NCHUNK, C), jnp.int32),
            *[pltpu.VMEM((C, D), jnp.float32) for _ in range(NB)],
            pltpu.VMEM_SHARED((N_MOL, D), jnp.float32),
            *[pltpu.SemaphoreType.DMA for _ in range(2 * NB)],
        ],
    )
    def seg_sum(feat_hbm, mi_hbm, pf_hbm, z_hbm, out_hbm,
                pf_v, mi_v, pm_v, fv0, fv1, fv2, fv3, fv4, fv5, acc_sh,
                li0, li1, li2, li3, li4, li5, ai0, ai1, ai2, ai3, ai4, ai5):
        core = lax.axis_index("c")
        sub = lax.axis_index("s")
        wid = core * NS + sub
        base = wid * PT
        bufs = (fv0, fv1, fv2, fv3, fv4, fv5)
        lsems = (li0, li1, li2, li3, li4, li5)
        asems = (ai0, ai1, ai2, ai3, ai4, ai5)

        def start_load(j, b):
            pltpu.async_copy(feat_hbm.at[pl.ds(base + j * C, C)], bufs[b], lsems[b])

        def wait_load(b):
            pltpu.make_async_copy(feat_hbm.at[pl.ds(0, C)], bufs[b], lsems[b]).wait()

        def start_add(j, b):
            pltpu.async_copy(bufs[b], acc_sh.at[pm_v.at[j]], asems[b], add=True)

        def wait_add(j, b):
            pltpu.make_async_copy(bufs[b], acc_sh.at[pm_v.at[j]], asems[b]).wait()

        for _j in range(NB - 1):
            start_load(_j, _j)

        pltpu.sync_copy(pf_hbm.at[pl.ds(base, PT)], pf_v)
        pltpu.sync_copy(mi_hbm, mi_v)

        @pl.when(sub == 0)
        def _():
            pltpu.sync_copy(z_hbm, acc_sh)

        def gather_body(j, carry):
            r0 = j * C
            for k in range(C // L):
                idx = pf_v[pl.ds(r0 + k * L, L)]
                pm_v[j, pl.ds(k * L, L)] = plsc.load_gather(mi_v, [idx])
            return carry

        lax.fori_loop(0, NCHUNK, gather_body, 0)

        plsc.subcore_barrier()

        def add_body(jj, carry):
            for b in range(NB):
                j = NB * jj + b
                wait_load(b)
                start_add(j, b)

                @pl.when(j >= 1)
                def _():
                    wait_add(j - 1, (b - 1) % NB)

                @pl.when(j + (NB - 1) < NCHUNK)
                def _():
                    start_load(j + (NB - 1), (b + NB - 1) % NB)

            return carry

        NFULL = NCHUNK // NB
        lax.fori_loop(0, NFULL, add_body, 0)

        for j in range(NFULL * NB, NCHUNK):
            b = j % NB
            wait_load(b)
            start_add(j, b)
            wait_add(j - 1, (b - 1) % NB)
        wait_add(NCHUNK - 1, (NCHUNK - 1) % NB)

        plsc.subcore_barrier()

        rows = N_MOL // NS
        pltpu.sync_copy(acc_sh.at[pl.ds(sub * rows, rows)],
                        out_hbm.at[core, pl.ds(sub * rows, rows)])

    return seg_sum(pairfeatures, mol_index, pair_first, zeros)


def _tc_partial(mol_index, pair_first, pairfeatures):
    def body(mi_ref, pf_ref, feat_ref, out_ref, lo_ref, hi_ref):
        i = pl.program_id(0)

        @pl.when(i == 0)
        def _():
            m = lax.broadcasted_iota(jnp.int32, (N_MOL, 128), 0)

            def row(r, carry):
                lo, hi = carry
                a = mi_ref[pl.ds(pl.multiple_of(r * 128, 128), 128)]
                a = a.reshape(1, 128)
                lo = lo + jnp.sum((a < m).astype(jnp.int32), axis=1,
                                  keepdims=True)
                hi = hi + jnp.sum((a <= m).astype(jnp.int32), axis=1,
                                  keepdims=True)
                return lo, hi

            lo0 = jnp.zeros((N_MOL, 1), jnp.int32)
            hi0 = jnp.zeros((N_MOL, 1), jnp.int32)
            lo, hi = lax.fori_loop(0, N_ATOMS // 128, row, (lo0, hi0))
            mt = lax.broadcasted_iota(jnp.int32, (N_MOL, 16), 0)
            at = mi_ref[pl.ds(N_ATOMS - 16, 16)].reshape(1, 16)
            lo = lo + jnp.sum((at < mt).astype(jnp.int32), axis=1,
                              keepdims=True)
            hi = hi + jnp.sum((at <= mt).astype(jnp.int32), axis=1,
                              keepdims=True)
            lo_ref[...] = lo
            hi_ref[...] = hi
            out_ref[...] = jnp.zeros_like(out_ref)

        off = pl.multiple_of((TC_BLK0 + i) * BP, BP)
        pfb = pf_ref[pl.ds(off, BP)].reshape(1, BP).astype(jnp.int16)
        lo = lo_ref[...].astype(jnp.int16)
        hi = hi_ref[...].astype(jnp.int16)
        oh = ((pfb >= lo) & (pfb < hi)).astype(jnp.bfloat16)
        fb = feat_ref[...].astype(jnp.bfloat16)
        out_ref[...] += lax.dot_general(
            oh, fb, (((1,), (0,)), ((), ())),
            preferred_element_type=jnp.float32)

    return pl.pallas_call(
        body,
        grid=(TC_NBLK,),
        in_specs=[
            pl.BlockSpec((N_ATOMS,), lambda i: (0,)),
            pl.BlockSpec((N_PAIRS,), lambda i: (0,)),
            pl.BlockSpec((BP, D), lambda i: (TC_BLK0 + i, 0)),
        ],
        out_specs=pl.BlockSpec((N_MOL, D), lambda i: (0, 0)),
        out_shape=jax.ShapeDtypeStruct((N_MOL, D), jnp.float32),
        scratch_shapes=[
            pltpu.VMEM((N_MOL, 1), jnp.int32),
            pltpu.VMEM((N_MOL, 1), jnp.int32),
        ],
    )(mol_index, pair_first, pairfeatures)


def _combine(sc_partials, tc_partial):
    def body(p_ref, t_ref, o_ref):
        o_ref[...] = p_ref[0] + p_ref[1] + t_ref[...]

    return pl.pallas_call(
        body,
        out_shape=jax.ShapeDtypeStruct((N_MOL, D), jnp.float32),
    )(sc_partials, tc_partial)


def kernel(pairfeatures, mol_index, n_molecules, pair_first):
    mi32 = mol_index.astype(jnp.int32)
    pf32 = pair_first.astype(jnp.int32)
    zeros = jnp.zeros((N_MOL, D), dtype=jnp.float32)
    sc_partials = _sc_segment_sum(pairfeatures, mi32, pf32, zeros)
    tc_partial = _tc_partial(mi32, pf32, pairfeatures)
    return _combine(sc_partials, tc_partial)
